# baseline (speedup 1.0000x reference)
import functools

import jax
import jax.numpy as jnp
import numpy as np
from jax import lax
from jax.experimental import pallas as pl
from jax.experimental.pallas import tpu as pltpu
from jax.experimental.pallas import tpu_sc as plsc

D = 16
MIN_NORM = 1e-15
BALL_EPS = 4e-3
ATANH_CLIP = 1.0 - 1e-7
LN2 = 0.6931471805599453

N_CORES, N_SUB = 2, 16
NW = N_CORES * N_SUB


def _widx():
    return lax.axis_index("c") * N_SUB + lax.axis_index("s")


def _rsqrt(s):
    bits = plsc.bitcast(s, jnp.int32)
    r = plsc.bitcast(np.int32(0x5F3759DF) - (bits >> 1), jnp.float32)
    for _ in range(3):
        r = r * (1.5 - (0.5 * s * r) * r)
    return r


def _ln(y):
    bits = plsc.bitcast(y, jnp.int32)
    e = (bits >> 23) - 127
    m = plsc.bitcast((bits & np.int32(0x007FFFFF)) | np.int32(0x3F800000),
                     jnp.float32)
    t = (m - 1.0) / (m + 1.0)
    t2 = t * t
    p = t * (2.0 + t2 * (2.0 / 3.0 + t2 * (2.0 / 5.0 + t2 * (2.0 / 7.0
                                                             + t2 * (2.0 / 9.0)))))
    return e.astype(jnp.float32) * LN2 + p


def _transpose_cols(buf, rows):
    return [
        plsc.load_gather(buf, [rows, jnp.full((16,), d, jnp.int32)])
        for d in range(D)
    ]


def _sc_logmap_table_t(emb_t):
    _, v = emb_t.shape
    ch = 1600
    n_chunks = v // ch
    orows = ch * D // 128
    mesh = plsc.VectorSubcoreMesh(core_axis_name="c", subcore_axis_name="s")

    @functools.partial(
        pl.kernel,
        mesh=mesh,
        compiler_params=pltpu.CompilerParams(
            use_tc_tiling_on_sc=False, needs_layout_passes=False),
        out_type=jax.ShapeDtypeStruct((v * D // 128, 128), jnp.float32),
        scratch_types=[
            pltpu.VMEM((2, D, ch), jnp.float32),
            pltpu.VMEM((2, orows, 128), jnp.float32),
            pltpu.SemaphoreType.DMA,
            pltpu.SemaphoreType.DMA,
            pltpu.SemaphoreType.DMA,
            pltpu.SemaphoreType.DMA,
        ],
    )
    def k(embt_hbm, tang_hbm, ebuf, obuf, si0, si1, so0, so1):
        wid = _widx()
        sin = (si0, si1)
        sout = (so0, so1)
        n_mine = (n_chunks - wid + NW - 1) // NW

        def start_in(i, b):
            c0 = (wid + i * NW) * ch
            for d in range(D):
                pltpu.async_copy(
                    embt_hbm.at[d, pl.ds(c0, ch)], ebuf.at[b, d], sin[b])

        def wait_in(b):
            pltpu.make_async_copy(
                embt_hbm.at[pl.ds(0, D), pl.ds(0, ch)], ebuf.at[b], sin[b]
            ).wait()

        def wait_out(b):
            pltpu.make_async_copy(
                tang_hbm.at[pl.ds(0, orows)], obuf.at[b], sout[b]
            ).wait()

        start_in(0, 0)
        start_in(1, 1)

        iot = lax.iota(jnp.int32, 16)
        rof = iot >> 3
        cbase = (iot & 7) << 4

        def process_chunk(i, b):
            wait_in(b)

            @pl.when(i >= 2)
            def _():
                wait_out(b)

            def blk(j, c2):
                cols = [ebuf[b, d, pl.ds(j * 16, 16)] for d in range(D)]
                s = cols[0] * cols[0]
                for d in range(1, D):
                    s = s + cols[d] * cols[d]
                r = _rsqrt(s)
                n = jnp.minimum(s * r, ATANH_CLIP)
                f = (0.5 * _ln((1.0 + n) / (1.0 - n))) * r
                orow = 2 * j + rof
                for d in range(D):
                    plsc.store_scatter(
                        obuf.at[b], [orow, cbase + d], cols[d] * f)
                return c2

            lax.fori_loop(0, ch // 16, blk, 0)
            r0 = (wid + i * NW) * orows
            pltpu.async_copy(
                obuf.at[b], tang_hbm.at[pl.ds(r0, orows)], sout[b])

            @pl.when(i + 2 < n_mine)
            def _():
                start_in(i + 2, b)

        def step(s0, carry):
            for b in range(2):
                process_chunk(s0 * 2 + b, b)
            return carry

        lax.fori_loop(0, n_mine // 2, step, 0)
        for b in range(2):
            @pl.when((n_mine % 2 == 1) & ((n_mine - 1) % 2 == b))
            def _(b=b):
                process_chunk(n_mine - 1, b)
        wait_out(0)
        wait_out(1)

    return k(emb_t)


def _sc_gather_sum_finalize(tang, ids3d, batch, seq_len):
    ex_w = batch // NW
    ech = 16
    nstep = ex_w // ech
    rows_c = ech * seq_len
    ksub = rows_c // 128
    mesh = plsc.VectorSubcoreMesh(core_axis_name="c", subcore_axis_name="s")

    @functools.partial(
        pl.kernel,
        mesh=mesh,
        compiler_params=pltpu.CompilerParams(
            use_tc_tiling_on_sc=False, needs_layout_passes=False),
        out_type=jax.ShapeDtypeStruct((batch, D), jnp.float32),
        scratch_types=[
            pltpu.VMEM((2, ksub, 128), jnp.int32),
            pltpu.VMEM((2, rows_c, D), jnp.float32),
            pltpu.VMEM((ex_w, D), jnp.float32),
            pltpu.SemaphoreType.DMA,
            pltpu.SemaphoreType.DMA,
        ],
    )
    def k(tang_hbm, ids_hbm, out_hbm, idx_v, rows_v, out_v, sem0, sem1):
        sems = (sem0, sem1)
        wid = _widx()

        def start_load(s, b):
            chunk = wid * nstep + s
            pltpu.sync_copy(ids_hbm.at[chunk], idx_v.at[b])
            for j in range(ksub):
                pltpu.async_copy(
                    tang_hbm.at[idx_v.at[b, j]],
                    rows_v.at[b, pl.ds(j * 128, 128)],
                    sems[b],
                )

        def wait_rows(b):
            pltpu.make_async_copy(
                tang_hbm.at[pl.ds(0, rows_c)], rows_v.at[b], sems[b]
            ).wait()

        def sum_example(b, base):
            zero = jnp.zeros((D,), jnp.float32)

            def tbody(i, accs):
                a0, a1, a2, a3 = accs
                o = base + i * 8
                a0 = a0 + rows_v[b, o]
                a1 = a1 + rows_v[b, o + 1]
                a2 = a2 + rows_v[b, o + 2]
                a3 = a3 + rows_v[b, o + 3]
                a0 = a0 + rows_v[b, o + 4]
                a1 = a1 + rows_v[b, o + 5]
                a2 = a2 + rows_v[b, o + 6]
                a3 = a3 + rows_v[b, o + 7]
                return a0, a1, a2, a3

            a0, a1, a2, a3 = lax.fori_loop(
                0, seq_len // 8, tbody, (zero, zero, zero, zero)
            )
            return (a0 + a1) + (a2 + a3)

        start_load(0, 0)
        start_load(1, 1)

        def step(s0, carry):
            for b in range(2):
                s = s0 * 2 + b
                wait_rows(b)
                for e in range(ech):
                    out_v[s * ech + e] = sum_example(b, e * seq_len)

                @pl.when(s + 2 < nstep)
                def _():
                    start_load(s + 2, b)
            return carry

        lax.fori_loop(0, nstep // 2, step, 0)

        def fin(bb, carry):
            rows = bb * 16 + lax.iota(jnp.int32, 16)
            cols = _transpose_cols(out_v, rows)
            mean = [c * (1.0 / seq_len) for c in cols]
            s = mean[0] * mean[0]
            for d in range(1, D):
                s = s + mean[d] * mean[d]
            r = _rsqrt(s)
            n = s * r
            e2 = jnp.exp(-2.0 * n)
            th = (1.0 - e2) / (1.0 + e2)
            f = jnp.minimum(th, 1.0 - BALL_EPS) * r
            for d in range(D):
                plsc.store_scatter(
                    out_v, [rows, jnp.full((16,), d, jnp.int32)],
                    mean[d] * f)
            return carry

        lax.fori_loop(0, ex_w // 16, fin, 0)
        pltpu.sync_copy(out_v, out_hbm.at[pl.ds(wid * ex_w, ex_w)])

    return k(tang, ids3d)


def kernel(emb, input_ids):
    batch, seq_len = input_ids.shape
    v, d = emb.shape
    tang128 = _sc_logmap_table_t(emb.T)
    tang = tang128.reshape(v, d)
    n_chunks = batch // (NW * 16) * NW
    ksub = 16 * seq_len // 128
    ids3d = input_ids.astype(jnp.int32).reshape(n_chunks, ksub, 128)
    return _sc_gather_sum_finalize(tang, ids3d, batch, seq_len)


# speedup vs baseline: 4.5024x; 4.5024x over previous
import functools

import jax
import jax.numpy as jnp
import numpy as np
from jax import lax
from jax.experimental import pallas as pl
from jax.experimental.pallas import tpu as pltpu
from jax.experimental.pallas import tpu_sc as plsc

D = 16
MIN_NORM = 1e-15
BALL_EPS = 4e-3
ATANH_CLIP = 1.0 - 1e-7
LN2 = 0.6931471805599453

N_CORES, N_SUB = 2, 16
NW = N_CORES * N_SUB


def _widx():
    return lax.axis_index("c") * N_SUB + lax.axis_index("s")


def _rsqrt(s):
    bits = plsc.bitcast(s, jnp.int32)
    r = plsc.bitcast(np.int32(0x5F3759DF) - (bits >> 1), jnp.float32)
    for _ in range(3):
        r = r * (1.5 - (0.5 * s * r) * r)
    return r


def _ln(y):
    bits = plsc.bitcast(y, jnp.int32)
    e = (bits >> 23) - 127
    m = plsc.bitcast((bits & np.int32(0x007FFFFF)) | np.int32(0x3F800000),
                     jnp.float32)
    t = (m - 1.0) / (m + 1.0)
    t2 = t * t
    p = t * (2.0 + t2 * (2.0 / 3.0 + t2 * (2.0 / 5.0 + t2 * (2.0 / 7.0
                                                             + t2 * (2.0 / 9.0)))))
    return e.astype(jnp.float32) * LN2 + p


def _transpose_cols(buf, rows):
    return [
        plsc.load_gather(buf, [rows, jnp.full((16,), d, jnp.int32)])
        for d in range(D)
    ]


def _logmap_factor(s):
    r = _rsqrt(s)
    n = jnp.minimum(s * r, ATANH_CLIP)
    return (0.5 * _ln((1.0 + n) / (1.0 - n))) * r


def _sc_logmap_table_tiled(emb_t, tail128):
    _, v = emb_t.shape
    ch = 1536
    vmain = (v // 128) * 128
    n_chunks = vmain // ch
    orows = ch * D // 128
    mesh = plsc.VectorSubcoreMesh(core_axis_name="c", subcore_axis_name="s")

    @functools.partial(
        pl.kernel,
        mesh=mesh,
        compiler_params=pltpu.CompilerParams(
            use_tc_tiling_on_sc=True, needs_layout_passes=False),
        out_type=jax.ShapeDtypeStruct((v * D // 128, 128), jnp.float32),
        scratch_types=[
            pltpu.VMEM((2, D, ch), jnp.float32),
            pltpu.VMEM((2, orows, 128), jnp.float32),
            pltpu.VMEM((8, 128), jnp.float32),
            pltpu.VMEM((8, 128), jnp.float32),
            pltpu.SemaphoreType.DMA,
            pltpu.SemaphoreType.DMA,
            pltpu.SemaphoreType.DMA,
            pltpu.SemaphoreType.DMA,
        ],
    )
    def k(embt_hbm, tail_hbm, tang_hbm, ebuf, obuf, tailv, otail,
          si0, si1, so0, so1):
        wid = _widx()
        sin = (si0, si1)
        sout = (so0, so1)
        n_mine = (n_chunks - wid + NW - 1) // NW

        iot = lax.iota(jnp.int32, 16)
        rof = iot >> 3
        cbase = (iot & 7) << 4

        def start_in(i, b):
            c0 = (wid + i * NW) * ch
            pltpu.async_copy(
                embt_hbm.at[:, pl.ds(c0, ch)], ebuf.at[b], sin[b])

        def wait_in(b):
            pltpu.make_async_copy(
                embt_hbm.at[:, pl.ds(0, ch)], ebuf.at[b], sin[b]).wait()

        def wait_out(b):
            pltpu.make_async_copy(
                tang_hbm.at[pl.ds(0, orows)], obuf.at[b], sout[b]).wait()

        start_in(0, 0)
        start_in(1, 1)

        def process_chunk(i, b):
            wait_in(b)

            @pl.when(i >= 2)
            def _():
                wait_out(b)

            def blk(j, c2):
                cols = [ebuf[b, d, pl.ds(j * 16, 16)] for d in range(D)]
                s = cols[0] * cols[0]
                for d in range(1, D):
                    s = s + cols[d] * cols[d]
                f = _logmap_factor(s)
                orow = 2 * j + rof
                for d in range(D):
                    plsc.store_scatter(
                        obuf.at[b], [orow, cbase + d], cols[d] * f)
                return c2

            lax.fori_loop(0, ch // 16, blk, 0)
            r0 = (wid + i * NW) * orows
            pltpu.async_copy(
                obuf.at[b], tang_hbm.at[pl.ds(r0, orows)], sout[b])

            @pl.when(i + 2 < n_mine)
            def _():
                start_in(i + 2, b)

        def step(s0, carry):
            for b in range(2):
                process_chunk(s0 * 2 + b, b)
            return carry

        lax.fori_loop(0, n_mine // 2, step, 0)
        for b in range(2):
            @pl.when((n_mine % 2 == 1) & ((n_mine - 1) % 2 == b))
            def _(b=b):
                process_chunk(n_mine - 1, b)
        wait_out(0)
        wait_out(1)

        @pl.when(wid == 0)
        def _():
            pltpu.sync_copy(tail_hbm, tailv)

            def tblk(j, c2):
                orow = 2 * j + rof
                cols = [
                    plsc.load_gather(tailv, [orow, cbase + d])
                    for d in range(D)
                ]
                s = cols[0] * cols[0]
                for d in range(1, D):
                    s = s + cols[d] * cols[d]
                f = _logmap_factor(s)
                for d in range(D):
                    plsc.store_scatter(otail, [orow, cbase + d], cols[d] * f)
                return c2

            lax.fori_loop(0, 4, tblk, 0)
            pltpu.sync_copy(otail, tang_hbm.at[pl.ds(vmain * D // 128, 8)])

    return k(emb_t, tail128)


def _sc_logmap_table_t(emb_t):
    _, v = emb_t.shape
    ch = 1600
    n_chunks = v // ch
    orows = ch * D // 128
    mesh = plsc.VectorSubcoreMesh(core_axis_name="c", subcore_axis_name="s")

    @functools.partial(
        pl.kernel,
        mesh=mesh,
        compiler_params=pltpu.CompilerParams(
            use_tc_tiling_on_sc=False, needs_layout_passes=False),
        out_type=jax.ShapeDtypeStruct((v * D // 128, 128), jnp.float32),
        scratch_types=[
            pltpu.VMEM((2, D, ch), jnp.float32),
            pltpu.VMEM((2, orows, 128), jnp.float32),
            pltpu.SemaphoreType.DMA,
            pltpu.SemaphoreType.DMA,
            pltpu.SemaphoreType.DMA,
            pltpu.SemaphoreType.DMA,
        ],
    )
    def k(embt_hbm, tang_hbm, ebuf, obuf, si0, si1, so0, so1):
        wid = _widx()
        sin = (si0, si1)
        sout = (so0, so1)
        n_mine = (n_chunks - wid + NW - 1) // NW

        def start_in(i, b):
            c0 = (wid + i * NW) * ch
            for d in range(D):
                pltpu.async_copy(
                    embt_hbm.at[d, pl.ds(c0, ch)], ebuf.at[b, d], sin[b])

        def wait_in(b):
            pltpu.make_async_copy(
                embt_hbm.at[pl.ds(0, D), pl.ds(0, ch)], ebuf.at[b], sin[b]
            ).wait()

        def wait_out(b):
            pltpu.make_async_copy(
                tang_hbm.at[pl.ds(0, orows)], obuf.at[b], sout[b]
            ).wait()

        start_in(0, 0)
        start_in(1, 1)

        iot = lax.iota(jnp.int32, 16)
        rof = iot >> 3
        cbase = (iot & 7) << 4

        def process_chunk(i, b):
            wait_in(b)

            @pl.when(i >= 2)
            def _():
                wait_out(b)

            def blk(j, c2):
                cols = [ebuf[b, d, pl.ds(j * 16, 16)] for d in range(D)]
                s = cols[0] * cols[0]
                for d in range(1, D):
                    s = s + cols[d] * cols[d]
                r = _rsqrt(s)
                n = jnp.minimum(s * r, ATANH_CLIP)
                f = (0.5 * _ln((1.0 + n) / (1.0 - n))) * r
                orow = 2 * j + rof
                for d in range(D):
                    plsc.store_scatter(
                        obuf.at[b], [orow, cbase + d], cols[d] * f)
                return c2

            lax.fori_loop(0, ch // 16, blk, 0)
            r0 = (wid + i * NW) * orows
            pltpu.async_copy(
                obuf.at[b], tang_hbm.at[pl.ds(r0, orows)], sout[b])

            @pl.when(i + 2 < n_mine)
            def _():
                start_in(i + 2, b)

        def step(s0, carry):
            for b in range(2):
                process_chunk(s0 * 2 + b, b)
            return carry

        lax.fori_loop(0, n_mine // 2, step, 0)
        for b in range(2):
            @pl.when((n_mine % 2 == 1) & ((n_mine - 1) % 2 == b))
            def _(b=b):
                process_chunk(n_mine - 1, b)
        wait_out(0)
        wait_out(1)

    return k(emb_t)


def _sc_gather_sum_finalize(tang, ids3d, batch, seq_len):
    ex_w = batch // NW
    ech = 16
    nstep = ex_w // ech
    rows_c = ech * seq_len
    ksub = rows_c // 128
    mesh = plsc.VectorSubcoreMesh(core_axis_name="c", subcore_axis_name="s")

    @functools.partial(
        pl.kernel,
        mesh=mesh,
        compiler_params=pltpu.CompilerParams(
            use_tc_tiling_on_sc=False, needs_layout_passes=False),
        out_type=jax.ShapeDtypeStruct((batch, D), jnp.float32),
        scratch_types=[
            pltpu.VMEM((2, ksub, 128), jnp.int32),
            pltpu.VMEM((2, rows_c, D), jnp.float32),
            pltpu.VMEM((ex_w, D), jnp.float32),
            pltpu.SemaphoreType.DMA,
            pltpu.SemaphoreType.DMA,
        ],
    )
    def k(tang_hbm, ids_hbm, out_hbm, idx_v, rows_v, out_v, sem0, sem1):
        sems = (sem0, sem1)
        wid = _widx()

        def start_load(s, b):
            chunk = wid * nstep + s
            pltpu.sync_copy(ids_hbm.at[chunk], idx_v.at[b])
            for j in range(ksub):
                pltpu.async_copy(
                    tang_hbm.at[idx_v.at[b, j]],
                    rows_v.at[b, pl.ds(j * 128, 128)],
                    sems[b],
                )

        def wait_rows(b):
            pltpu.make_async_copy(
                tang_hbm.at[pl.ds(0, rows_c)], rows_v.at[b], sems[b]
            ).wait()

        def sum_example(b, base):
            zero = jnp.zeros((D,), jnp.float32)

            def tbody(i, accs):
                a0, a1, a2, a3 = accs
                o = base + i * 8
                a0 = a0 + rows_v[b, o]
                a1 = a1 + rows_v[b, o + 1]
                a2 = a2 + rows_v[b, o + 2]
                a3 = a3 + rows_v[b, o + 3]
                a0 = a0 + rows_v[b, o + 4]
                a1 = a1 + rows_v[b, o + 5]
                a2 = a2 + rows_v[b, o + 6]
                a3 = a3 + rows_v[b, o + 7]
                return a0, a1, a2, a3

            a0, a1, a2, a3 = lax.fori_loop(
                0, seq_len // 8, tbody, (zero, zero, zero, zero)
            )
            return (a0 + a1) + (a2 + a3)

        start_load(0, 0)
        start_load(1, 1)

        def step(s0, carry):
            for b in range(2):
                s = s0 * 2 + b
                wait_rows(b)
                for e in range(ech):
                    out_v[s * ech + e] = sum_example(b, e * seq_len)

                @pl.when(s + 2 < nstep)
                def _():
                    start_load(s + 2, b)
            return carry

        lax.fori_loop(0, nstep // 2, step, 0)

        def fin(bb, carry):
            rows = bb * 16 + lax.iota(jnp.int32, 16)
            cols = _transpose_cols(out_v, rows)
            mean = [c * (1.0 / seq_len) for c in cols]
            s = mean[0] * mean[0]
            for d in range(1, D):
                s = s + mean[d] * mean[d]
            r = _rsqrt(s)
            n = s * r
            e2 = jnp.exp(-2.0 * n)
            th = (1.0 - e2) / (1.0 + e2)
            f = jnp.minimum(th, 1.0 - BALL_EPS) * r
            for d in range(D):
                plsc.store_scatter(
                    out_v, [rows, jnp.full((16,), d, jnp.int32)],
                    mean[d] * f)
            return carry

        lax.fori_loop(0, ex_w // 16, fin, 0)
        pltpu.sync_copy(out_v, out_hbm.at[pl.ds(wid * ex_w, ex_w)])

    return k(tang, ids3d)


def kernel(emb, input_ids):
    batch, seq_len = input_ids.shape
    v, d = emb.shape
    vmain = (v // 128) * 128
    tail128 = emb[vmain:].reshape(8, 128)
    tang128 = _sc_logmap_table_tiled(emb.T, tail128)
    tang = tang128.reshape(v, d)
    n_chunks = batch // (NW * 16) * NW
    ksub = 16 * seq_len // 128
    ids3d = input_ids.astype(jnp.int32).reshape(n_chunks, ksub, 128)
    return _sc_gather_sum_finalize(tang, ids3d, batch, seq_len)


# speedup vs baseline: 4.6318x; 1.0287x over previous
import functools

import jax
import jax.numpy as jnp
import numpy as np
from jax import lax
from jax.experimental import pallas as pl
from jax.experimental.pallas import tpu as pltpu
from jax.experimental.pallas import tpu_sc as plsc

D = 16
MIN_NORM = 1e-15
BALL_EPS = 4e-3
ATANH_CLIP = 1.0 - 1e-7
LN2 = 0.6931471805599453

N_CORES, N_SUB = 2, 16
NW = N_CORES * N_SUB


def _widx():
    return lax.axis_index("c") * N_SUB + lax.axis_index("s")


def _rsqrt(s):
    bits = plsc.bitcast(s, jnp.int32)
    r = plsc.bitcast(np.int32(0x5F3759DF) - (bits >> 1), jnp.float32)
    for _ in range(2):
        r = r * (1.5 - (0.5 * s * r) * r)
    return r


def _ln(y):
    bits = plsc.bitcast(y, jnp.int32)
    e = (bits >> 23) - 127
    m = plsc.bitcast((bits & np.int32(0x007FFFFF)) | np.int32(0x3F800000),
                     jnp.float32)
    t = (m - 1.0) / (m + 1.0)
    t2 = t * t
    p = t * (2.0 + t2 * (2.0 / 3.0 + t2 * (2.0 / 5.0 + t2 * (2.0 / 7.0
                                                             + t2 * (2.0 / 9.0)))))
    return e.astype(jnp.float32) * LN2 + p


def _transpose_cols(buf, rows):
    return [
        plsc.load_gather(buf, [rows, jnp.full((16,), d, jnp.int32)])
        for d in range(D)
    ]


def _logmap_factor(s):
    r = _rsqrt(s)
    n = jnp.minimum(s * r, ATANH_CLIP)
    return (0.5 * _ln((1.0 + n) / (1.0 - n))) * r


def _sc_logmap_table_tiled(emb_t, tail128):
    _, v = emb_t.shape
    ch = 1536
    vmain = (v // 128) * 128
    n_chunks = vmain // ch
    orows = ch * D // 128
    mesh = plsc.VectorSubcoreMesh(core_axis_name="c", subcore_axis_name="s")

    @functools.partial(
        pl.kernel,
        mesh=mesh,
        compiler_params=pltpu.CompilerParams(
            use_tc_tiling_on_sc=True, needs_layout_passes=False),
        out_type=jax.ShapeDtypeStruct((v * D // 128, 128), jnp.float32),
        scratch_types=[
            pltpu.VMEM((2, D, ch), jnp.float32),
            pltpu.VMEM((2, orows, 128), jnp.float32),
            pltpu.VMEM((8, 128), jnp.float32),
            pltpu.VMEM((8, 128), jnp.float32),
            pltpu.SemaphoreType.DMA,
            pltpu.SemaphoreType.DMA,
            pltpu.SemaphoreType.DMA,
            pltpu.SemaphoreType.DMA,
        ],
    )
    def k(embt_hbm, tail_hbm, tang_hbm, ebuf, obuf, tailv, otail,
          si0, si1, so0, so1):
        wid = _widx()
        sin = (si0, si1)
        sout = (so0, so1)
        n_mine = (n_chunks - wid + NW - 1) // NW

        iot = lax.iota(jnp.int32, 16)
        rof = iot >> 3
        cbase = (iot & 7) << 4

        def start_in(i, b):
            c0 = (wid + i * NW) * ch
            pltpu.async_copy(
                embt_hbm.at[:, pl.ds(c0, ch)], ebuf.at[b], sin[b])

        def wait_in(b):
            pltpu.make_async_copy(
                embt_hbm.at[:, pl.ds(0, ch)], ebuf.at[b], sin[b]).wait()

        def wait_out(b):
            pltpu.make_async_copy(
                tang_hbm.at[pl.ds(0, orows)], obuf.at[b], sout[b]).wait()

        start_in(0, 0)
        start_in(1, 1)

        def process_chunk(i, b):
            wait_in(b)

            @pl.when(i >= 2)
            def _():
                wait_out(b)

            @plsc.parallel_loop(0, ch // 16, 1, unroll=2)
            def blk(j):
                c0 = ebuf[b, 0, pl.ds(j * 16, 16)]
                s = c0 * c0
                for d in range(1, D):
                    c = ebuf[b, d, pl.ds(j * 16, 16)]
                    s = s + c * c
                f = _logmap_factor(s)
                orow = 2 * j + rof
                for d in range(D):
                    c = ebuf[b, d, pl.ds(j * 16, 16)]
                    plsc.store_scatter(
                        obuf.at[b], [orow, cbase + d], c * f)
            r0 = (wid + i * NW) * orows
            pltpu.async_copy(
                obuf.at[b], tang_hbm.at[pl.ds(r0, orows)], sout[b])

            @pl.when(i + 2 < n_mine)
            def _():
                start_in(i + 2, b)

        def step(s0, carry):
            for b in range(2):
                process_chunk(s0 * 2 + b, b)
            return carry

        lax.fori_loop(0, n_mine // 2, step, 0)
        for b in range(2):
            @pl.when((n_mine % 2 == 1) & ((n_mine - 1) % 2 == b))
            def _(b=b):
                process_chunk(n_mine - 1, b)
        wait_out(0)
        wait_out(1)

        @pl.when(wid == 0)
        def _():
            pltpu.sync_copy(tail_hbm, tailv)

            def tblk(j, c2):
                orow = 2 * j + rof
                cols = [
                    plsc.load_gather(tailv, [orow, cbase + d])
                    for d in range(D)
                ]
                s = cols[0] * cols[0]
                for d in range(1, D):
                    s = s + cols[d] * cols[d]
                f = _logmap_factor(s)
                for d in range(D):
                    plsc.store_scatter(otail, [orow, cbase + d], cols[d] * f)
                return c2

            lax.fori_loop(0, 4, tblk, 0)
            pltpu.sync_copy(otail, tang_hbm.at[pl.ds(vmain * D // 128, 8)])

    return k(emb_t, tail128)


def _sc_logmap_table_t(emb_t):
    _, v = emb_t.shape
    ch = 1600
    n_chunks = v // ch
    orows = ch * D // 128
    mesh = plsc.VectorSubcoreMesh(core_axis_name="c", subcore_axis_name="s")

    @functools.partial(
        pl.kernel,
        mesh=mesh,
        compiler_params=pltpu.CompilerParams(
            use_tc_tiling_on_sc=False, needs_layout_passes=False),
        out_type=jax.ShapeDtypeStruct((v * D // 128, 128), jnp.float32),
        scratch_types=[
            pltpu.VMEM((2, D, ch), jnp.float32),
            pltpu.VMEM((2, orows, 128), jnp.float32),
            pltpu.SemaphoreType.DMA,
            pltpu.SemaphoreType.DMA,
            pltpu.SemaphoreType.DMA,
            pltpu.SemaphoreType.DMA,
        ],
    )
    def k(embt_hbm, tang_hbm, ebuf, obuf, si0, si1, so0, so1):
        wid = _widx()
        sin = (si0, si1)
        sout = (so0, so1)
        n_mine = (n_chunks - wid + NW - 1) // NW

        def start_in(i, b):
            c0 = (wid + i * NW) * ch
            for d in range(D):
                pltpu.async_copy(
                    embt_hbm.at[d, pl.ds(c0, ch)], ebuf.at[b, d], sin[b])

        def wait_in(b):
            pltpu.make_async_copy(
                embt_hbm.at[pl.ds(0, D), pl.ds(0, ch)], ebuf.at[b], sin[b]
            ).wait()

        def wait_out(b):
            pltpu.make_async_copy(
                tang_hbm.at[pl.ds(0, orows)], obuf.at[b], sout[b]
            ).wait()

        start_in(0, 0)
        start_in(1, 1)

        iot = lax.iota(jnp.int32, 16)
        rof = iot >> 3
        cbase = (iot & 7) << 4

        def process_chunk(i, b):
            wait_in(b)

            @pl.when(i >= 2)
            def _():
                wait_out(b)

            def blk(j, c2):
                cols = [ebuf[b, d, pl.ds(j * 16, 16)] for d in range(D)]
                s = cols[0] * cols[0]
                for d in range(1, D):
                    s = s + cols[d] * cols[d]
                r = _rsqrt(s)
                n = jnp.minimum(s * r, ATANH_CLIP)
                f = (0.5 * _ln((1.0 + n) / (1.0 - n))) * r
                orow = 2 * j + rof
                for d in range(D):
                    plsc.store_scatter(
                        obuf.at[b], [orow, cbase + d], cols[d] * f)
                return c2

            lax.fori_loop(0, ch // 16, blk, 0)
            r0 = (wid + i * NW) * orows
            pltpu.async_copy(
                obuf.at[b], tang_hbm.at[pl.ds(r0, orows)], sout[b])

            @pl.when(i + 2 < n_mine)
            def _():
                start_in(i + 2, b)

        def step(s0, carry):
            for b in range(2):
                process_chunk(s0 * 2 + b, b)
            return carry

        lax.fori_loop(0, n_mine // 2, step, 0)
        for b in range(2):
            @pl.when((n_mine % 2 == 1) & ((n_mine - 1) % 2 == b))
            def _(b=b):
                process_chunk(n_mine - 1, b)
        wait_out(0)
        wait_out(1)

    return k(emb_t)


def _sc_gather_sum_finalize(tang, ids3d, batch, seq_len):
    ex_w = batch // NW
    ech = 16
    nstep = ex_w // ech
    rows_c = ech * seq_len
    ksub = rows_c // 128
    mesh = plsc.VectorSubcoreMesh(core_axis_name="c", subcore_axis_name="s")

    @functools.partial(
        pl.kernel,
        mesh=mesh,
        compiler_params=pltpu.CompilerParams(
            use_tc_tiling_on_sc=False, needs_layout_passes=False),
        out_type=jax.ShapeDtypeStruct((batch, D), jnp.float32),
        scratch_types=[
            pltpu.VMEM((2, ksub, 128), jnp.int32),
            pltpu.VMEM((2, rows_c, D), jnp.float32),
            pltpu.VMEM((ex_w, D), jnp.float32),
            pltpu.SemaphoreType.DMA,
            pltpu.SemaphoreType.DMA,
        ],
    )
    def k(tang_hbm, ids_hbm, out_hbm, idx_v, rows_v, out_v, sem0, sem1):
        sems = (sem0, sem1)
        wid = _widx()

        def start_load(s, b):
            chunk = wid * nstep + s
            pltpu.sync_copy(ids_hbm.at[chunk], idx_v.at[b])
            for j in range(ksub):
                pltpu.async_copy(
                    tang_hbm.at[idx_v.at[b, j]],
                    rows_v.at[b, pl.ds(j * 128, 128)],
                    sems[b],
                )

        def wait_rows(b):
            pltpu.make_async_copy(
                tang_hbm.at[pl.ds(0, rows_c)], rows_v.at[b], sems[b]
            ).wait()

        def sum_example(b, base):
            zero = jnp.zeros((D,), jnp.float32)

            def tbody(i, accs):
                a0, a1, a2, a3 = accs
                o = base + i * 8
                a0 = a0 + rows_v[b, o]
                a1 = a1 + rows_v[b, o + 1]
                a2 = a2 + rows_v[b, o + 2]
                a3 = a3 + rows_v[b, o + 3]
                a0 = a0 + rows_v[b, o + 4]
                a1 = a1 + rows_v[b, o + 5]
                a2 = a2 + rows_v[b, o + 6]
                a3 = a3 + rows_v[b, o + 7]
                return a0, a1, a2, a3

            a0, a1, a2, a3 = lax.fori_loop(
                0, seq_len // 8, tbody, (zero, zero, zero, zero)
            )
            return (a0 + a1) + (a2 + a3)

        start_load(0, 0)
        start_load(1, 1)

        def step(s0, carry):
            for b in range(2):
                s = s0 * 2 + b
                wait_rows(b)
                for e in range(ech):
                    out_v[s * ech + e] = sum_example(b, e * seq_len)

                @pl.when(s + 2 < nstep)
                def _():
                    start_load(s + 2, b)
            return carry

        lax.fori_loop(0, nstep // 2, step, 0)

        def fin(bb, carry):
            rows = bb * 16 + lax.iota(jnp.int32, 16)
            cols = _transpose_cols(out_v, rows)
            mean = [c * (1.0 / seq_len) for c in cols]
            s = mean[0] * mean[0]
            for d in range(1, D):
                s = s + mean[d] * mean[d]
            r = _rsqrt(s)
            n = s * r
            e2 = jnp.exp(-2.0 * n)
            th = (1.0 - e2) / (1.0 + e2)
            f = jnp.minimum(th, 1.0 - BALL_EPS) * r
            for d in range(D):
                plsc.store_scatter(
                    out_v, [rows, jnp.full((16,), d, jnp.int32)],
                    mean[d] * f)
            return carry

        lax.fori_loop(0, ex_w // 16, fin, 0)
        pltpu.sync_copy(out_v, out_hbm.at[pl.ds(wid * ex_w, ex_w)])

    return k(tang, ids3d)


def kernel(emb, input_ids):
    batch, seq_len = input_ids.shape
    v, d = emb.shape
    vmain = (v // 128) * 128
    tail128 = emb[vmain:].reshape(8, 128)
    tang128 = _sc_logmap_table_tiled(emb.T, tail128)
    tang = tang128.reshape(v, d)
    n_chunks = batch // (NW * 16) * NW
    ksub = 16 * seq_len // 128
    ids3d = input_ids.astype(jnp.int32).reshape(n_chunks, ksub, 128)
    return _sc_gather_sum_finalize(tang, ids3d, batch, seq_len)
